# baseline (speedup 1.0000x reference)
import functools

import jax
import jax.numpy as jnp
from jax import lax
from jax.experimental import pallas as pl
from jax.experimental.pallas import tpu as pltpu
from jax.experimental.pallas import tpu_sc as plsc

_NC = 2
_NS = 16
_NW = _NC * _NS
_K = 128
_EPS_BN = 1e-5


_NBUF = 4


@functools.lru_cache(maxsize=None)
def _make_agg(n, d, c, halves=False):
    zr = ((n // _NS) + _K - 1) // _K * _K
    npad = _NS * zr
    ro = (n // _NS) // 8 * 8
    rem = n - ro * _NS

    stage = (npad + n) * d + _NS * (2 * _K * d + 16 * _K) <= 2097151
    assert stage or not halves
    spare = (2097151 - (npad + (n if stage else 0)) * d) // _NS
    nbuf = _NBUF
    while nbuf > 2 and spare - nbuf * _K * d < 8 * _K:
        nbuf //= 2
    per_tile = spare - nbuf * _K * d
    cb_max = per_tile // (2 * _K)
    cb = max(b for b in range(1, c + 1)
             if c % b == 0 and b % nbuf == 0 and b <= cb_max)

    mesh = plsc.VectorSubcoreMesh(core_axis_name="c", subcore_axis_name="s")

    def body(h_hbm, src_hbm, dst_hbm, out_hbm, sidx, didx, rows, acc,
             *rest):
        if stage:
            hsp, *sems = rest
        else:
            sems = rest
        cid = lax.axis_index("c")
        sid = lax.axis_index("s")
        w = sid if halves else cid * _NS + sid

        if stage:
            hin = h_hbm.at[cid] if halves else h_hbm
            pltpu.sync_copy(hin.at[pl.ds(sid * ro, ro)],
                            hsp.at[pl.ds(sid * ro, ro)])
            if rem:
                @pl.when(sid == _NS - 1)
                def _():
                    pltpu.sync_copy(hin.at[pl.ds(ro * _NS, rem)],
                                    hsp.at[pl.ds(ro * _NS, rem)])
            h_srcs = [hsp] * nbuf
        else:
            h_srcs = [h_hbm] * nbuf

        def zero_rows(i, carry):
            r = i // (d // 16)
            col = (i % (d // 16)) * 16
            rows[0, r, pl.ds(col, 16)] = jnp.zeros((16,), jnp.float32)
            return carry

        lax.fori_loop(0, _K * (d // 16), zero_rows, 0)
        for t in range(zr // _K):
            pltpu.sync_copy(rows.at[0], acc.at[pl.ds(sid * zr + t * _K, _K)])
        plsc.subcore_barrier()

        sems_g = sems

        def wait_g(b, ej):
            pltpu.make_async_copy(h_srcs[b].at[sidx.at[ej]], rows.at[b],
                                  sems_g[b]).wait()

        for sb in range(c // cb):
            pltpu.sync_copy(src_hbm.at[w, pl.ds(sb * cb, cb)], sidx)
            pltpu.sync_copy(dst_hbm.at[w, pl.ds(sb * cb, cb)], didx)
            for b in range(nbuf):
                pltpu.async_copy(h_srcs[b].at[sidx.at[b]], rows.at[b],
                                 sems_g[b])

            def step(g, carry):
                e0 = g * nbuf
                for b in range(nbuf):
                    ej = e0 + b
                    wait_g(b, ej)
                    pltpu.sync_copy(rows.at[b], acc.at[didx.at[ej]],
                                    add=True)

                    @pl.when(ej + nbuf < cb)
                    def _(b=b, ej=ej):
                        pltpu.async_copy(h_srcs[b].at[sidx.at[ej + nbuf]],
                                         rows.at[b], sems_g[b])
                return carry

            lax.fori_loop(0, cb // nbuf, step, 0)
        plsc.subcore_barrier()
        pltpu.sync_copy(acc.at[pl.ds(sid * ro, ro)],
                        out_hbm.at[cid, pl.ds(sid * ro, ro)])
        if rem:
            @pl.when(sid == _NS - 1)
            def _():
                pltpu.sync_copy(acc.at[pl.ds(ro * _NS, rem)],
                                out_hbm.at[cid, pl.ds(ro * _NS, rem)])

    return pl.kernel(
        body,
        out_type=jax.ShapeDtypeStruct((_NC, n, d), jnp.float32),
        mesh=mesh,
        compiler_params=pltpu.CompilerParams(use_tc_tiling_on_sc=False),
        scratch_types=[
            pltpu.VMEM((cb, _K), jnp.int32),
            pltpu.VMEM((cb, _K), jnp.int32),
            pltpu.VMEM((nbuf, _K, d), jnp.float32),
            pltpu.VMEM_SHARED((npad, d), jnp.float32),
        ] + ([pltpu.VMEM_SHARED((n, d), jnp.float32)] if stage else [])
          + [pltpu.SemaphoreType.DMA] * nbuf,
    )


def _mlp_core(eps, h, a, W1, b1, g, be, W2, b2, halves):
    agg = jnp.concatenate([a[0], a[1]], axis=-1) if halves else a[0] + a[1]
    rst = (1.0 + eps) * h + agg
    x = jnp.dot(rst, W1, preferred_element_type=jnp.float32) + b1
    mean = jnp.mean(x, axis=0, keepdims=True)
    xc = x - mean
    var = jnp.mean(xc * xc, axis=0, keepdims=True)
    x = xc * lax.rsqrt(var + _EPS_BN) * g + be
    x = jnp.maximum(x, 0.0)
    x = jnp.dot(x, W2, preferred_element_type=jnp.float32) + b2
    return jnp.maximum(x, 0.0)


def _mlp_body(halves, eps_ref, h_ref, a_ref, W1_ref, b1_ref, g_ref, be_ref,
              W2_ref, b2_ref, o_ref):
    o_ref[...] = _mlp_core(eps_ref[0], h_ref[...], a_ref[...], W1_ref[...],
                           b1_ref[...], g_ref[...], be_ref[...], W2_ref[...],
                           b2_ref[...], halves)


def _mlp_out_body(halves, eps_ref, h_ref, a_ref, W1_ref, b1_ref, g_ref,
                  be_ref, W2_ref, b2_ref, Wo_ref, bo_ref, o_ref):
    hh = _mlp_core(eps_ref[0], h_ref[...], a_ref[...], W1_ref[...],
                   b1_ref[...], g_ref[...], be_ref[...], W2_ref[...],
                   b2_ref[...], halves)
    o_ref[...] = jnp.dot(hh, Wo_ref[...],
                         preferred_element_type=jnp.float32) + bo_ref[...]


def _specs(n_vmem):
    return [pl.BlockSpec(memory_space=pltpu.SMEM)] + \
           [pl.BlockSpec(memory_space=pltpu.VMEM)] * n_vmem


def kernel(node_feat, edge_index, params):
    n = node_feat.shape[0]
    e = edge_index.shape[1]
    c = -(-e // (_NW * _K))
    c = -(-c // _NBUF) * _NBUF
    epad = _NW * c * _K
    src = edge_index[0]
    dst = edge_index[1]
    if epad > e:
        zr = ((n // _NS) + _K - 1) // _K * _K
        padrows = _NS * zr - n
        sent = n + jnp.arange(epad - e, dtype=jnp.int32) % padrows
        src = jnp.concatenate([src, jnp.zeros((epad - e,), jnp.int32)])
        dst = jnp.concatenate([dst, sent])
    src32 = src.reshape(_NW, c, _K)
    dst32 = dst.reshape(_NW, c, _K)
    src16 = src.reshape(_NS, _NC * c, _K)
    dst16 = dst.reshape(_NS, _NC * c, _K)

    def _fits(dd):
        zrq = ((n // _NS) + _K - 1) // _K * _K
        return (_NS * zrq + n) * dd + _NS * (2 * _K * dd + 16 * _K) <= 2097151

    h = node_feat
    layers = params['layers']
    for i, lp in enumerate(layers):
        d = h.shape[1]
        halves = (not _fits(d)) and d % 2 == 0 and _fits(d // 2)
        if halves:
            h2 = jnp.transpose(h.reshape(n, 2, d // 2), (1, 0, 2))
            parts = _make_agg(n, d // 2, _NC * c, True)(h2, src16, dst16)
        else:
            parts = _make_agg(n, d, c)(h, src32, dst32)
        hd = lp['W1'].shape[1]
        args = (lp['eps'].reshape(1), h, parts, lp['W1'],
                lp['b1'].reshape(1, -1), lp['gamma'].reshape(1, -1),
                lp['beta'].reshape(1, -1), lp['W2'], lp['b2'].reshape(1, -1))
        if i + 1 < len(layers):
            h = pl.pallas_call(
                functools.partial(_mlp_body, halves),
                out_shape=jax.ShapeDtypeStruct((n, hd), jnp.float32),
                in_specs=_specs(8),
            )(*args)
        else:
            out_d = params['Wo'].shape[1]
            h = pl.pallas_call(
                functools.partial(_mlp_out_body, halves),
                out_shape=jax.ShapeDtypeStruct((n, out_d), jnp.float32),
                in_specs=_specs(10),
            )(*args, params['Wo'], params['bo'].reshape(1, -1))
    return h


# speedup vs baseline: 1.0337x; 1.0337x over previous
import functools

import jax
import jax.numpy as jnp
from jax import lax
from jax.experimental import pallas as pl
from jax.experimental.pallas import tpu as pltpu
from jax.experimental.pallas import tpu_sc as plsc

_NC = 2
_NS = 16
_NW = _NC * _NS
_K = 128
_EPS_BN = 1e-5


_NBUF = 2


@functools.lru_cache(maxsize=None)
def _make_agg(n, d, c, halves=False):
    zr = ((n // _NS) + _K - 1) // _K * _K
    npad = _NS * zr
    ro = (n // _NS) // 8 * 8
    rem = n - ro * _NS

    stage = (npad + n) * d + _NS * (2 * _K * d + 16 * _K) <= 2097151
    assert stage or not halves
    spare = (2097151 - (npad + (n if stage else 0)) * d) // _NS
    nbuf = _NBUF
    while nbuf > 2 and spare - nbuf * _K * d < 8 * _K:
        nbuf //= 2
    per_tile = spare - nbuf * _K * d
    cb_max = per_tile // (2 * _K)
    cb = max(b for b in range(1, c + 1)
             if c % b == 0 and b % nbuf == 0 and b <= cb_max)

    mesh = plsc.VectorSubcoreMesh(core_axis_name="c", subcore_axis_name="s")

    def body(h_hbm, src_hbm, dst_hbm, out_hbm, sidx, didx, rows, acc,
             *rest):
        if stage:
            hsp, *sems = rest
        else:
            sems = rest
        cid = lax.axis_index("c")
        sid = lax.axis_index("s")
        w = sid if halves else cid * _NS + sid

        if stage:
            hin = h_hbm.at[cid] if halves else h_hbm
            pltpu.sync_copy(hin.at[pl.ds(sid * ro, ro)],
                            hsp.at[pl.ds(sid * ro, ro)])
            if rem:
                @pl.when(sid == _NS - 1)
                def _():
                    pltpu.sync_copy(hin.at[pl.ds(ro * _NS, rem)],
                                    hsp.at[pl.ds(ro * _NS, rem)])
            h_srcs = [hsp] * nbuf
        else:
            h_srcs = [h_hbm] * nbuf

        def zero_rows(i, carry):
            r = i // (d // 16)
            col = (i % (d // 16)) * 16
            rows[0, r, pl.ds(col, 16)] = jnp.zeros((16,), jnp.float32)
            return carry

        lax.fori_loop(0, _K * (d // 16), zero_rows, 0)
        for t in range(zr // _K):
            pltpu.sync_copy(rows.at[0], acc.at[pl.ds(sid * zr + t * _K, _K)])
        plsc.subcore_barrier()

        sems_g = sems

        def wait_g(b, ej):
            pltpu.make_async_copy(h_srcs[b].at[sidx.at[ej]], rows.at[b],
                                  sems_g[b]).wait()

        for sb in range(c // cb):
            pltpu.sync_copy(src_hbm.at[w, pl.ds(sb * cb, cb)], sidx)
            pltpu.sync_copy(dst_hbm.at[w, pl.ds(sb * cb, cb)], didx)
            for b in range(nbuf):
                pltpu.async_copy(h_srcs[b].at[sidx.at[b]], rows.at[b],
                                 sems_g[b])

            def step(g, carry):
                e0 = g * nbuf
                for b in range(nbuf):
                    ej = e0 + b
                    wait_g(b, ej)
                    pltpu.sync_copy(rows.at[b], acc.at[didx.at[ej]],
                                    add=True)

                    @pl.when(ej + nbuf < cb)
                    def _(b=b, ej=ej):
                        pltpu.async_copy(h_srcs[b].at[sidx.at[ej + nbuf]],
                                         rows.at[b], sems_g[b])
                return carry

            lax.fori_loop(0, cb // nbuf, step, 0)
        plsc.subcore_barrier()
        pltpu.sync_copy(acc.at[pl.ds(sid * ro, ro)],
                        out_hbm.at[cid, pl.ds(sid * ro, ro)])
        if rem:
            @pl.when(sid == _NS - 1)
            def _():
                pltpu.sync_copy(acc.at[pl.ds(ro * _NS, rem)],
                                out_hbm.at[cid, pl.ds(ro * _NS, rem)])

    return pl.kernel(
        body,
        out_type=jax.ShapeDtypeStruct((_NC, n, d), jnp.float32),
        mesh=mesh,
        compiler_params=pltpu.CompilerParams(use_tc_tiling_on_sc=False),
        scratch_types=[
            pltpu.VMEM((cb, _K), jnp.int32),
            pltpu.VMEM((cb, _K), jnp.int32),
            pltpu.VMEM((nbuf, _K, d), jnp.float32),
            pltpu.VMEM_SHARED((npad, d), jnp.float32),
        ] + ([pltpu.VMEM_SHARED((n, d), jnp.float32)] if stage else [])
          + [pltpu.SemaphoreType.DMA] * nbuf,
    )


def _mlp_core(eps, h, a, W1, b1, g, be, W2, b2, halves):
    agg = jnp.concatenate([a[0], a[1]], axis=-1) if halves else a[0] + a[1]
    rst = (1.0 + eps) * h + agg
    x = jnp.dot(rst, W1, preferred_element_type=jnp.float32) + b1
    mean = jnp.mean(x, axis=0, keepdims=True)
    xc = x - mean
    var = jnp.mean(xc * xc, axis=0, keepdims=True)
    x = xc * lax.rsqrt(var + _EPS_BN) * g + be
    x = jnp.maximum(x, 0.0)
    x = jnp.dot(x, W2, preferred_element_type=jnp.float32) + b2
    return jnp.maximum(x, 0.0)


def _mlp_body(halves, eps_ref, h_ref, a_ref, W1_ref, b1_ref, g_ref, be_ref,
              W2_ref, b2_ref, o_ref):
    o_ref[...] = _mlp_core(eps_ref[0], h_ref[...], a_ref[...], W1_ref[...],
                           b1_ref[...], g_ref[...], be_ref[...], W2_ref[...],
                           b2_ref[...], halves)


def _mlp_out_body(halves, eps_ref, h_ref, a_ref, W1_ref, b1_ref, g_ref,
                  be_ref, W2_ref, b2_ref, Wo_ref, bo_ref, o_ref):
    hh = _mlp_core(eps_ref[0], h_ref[...], a_ref[...], W1_ref[...],
                   b1_ref[...], g_ref[...], be_ref[...], W2_ref[...],
                   b2_ref[...], halves)
    o_ref[...] = jnp.dot(hh, Wo_ref[...],
                         preferred_element_type=jnp.float32) + bo_ref[...]


def _specs(n_vmem):
    return [pl.BlockSpec(memory_space=pltpu.SMEM)] + \
           [pl.BlockSpec(memory_space=pltpu.VMEM)] * n_vmem


def kernel(node_feat, edge_index, params):
    n = node_feat.shape[0]
    e = edge_index.shape[1]
    c = -(-e // (_NW * _K))
    c = -(-c // _NBUF) * _NBUF
    epad = _NW * c * _K
    src = edge_index[0]
    dst = edge_index[1]
    if epad > e:
        zr = ((n // _NS) + _K - 1) // _K * _K
        padrows = _NS * zr - n
        sent = n + jnp.arange(epad - e, dtype=jnp.int32) % padrows
        src = jnp.concatenate([src, jnp.zeros((epad - e,), jnp.int32)])
        dst = jnp.concatenate([dst, sent])
    src32 = src.reshape(_NW, c, _K)
    dst32 = dst.reshape(_NW, c, _K)
    src16 = src.reshape(_NS, _NC * c, _K)
    dst16 = dst.reshape(_NS, _NC * c, _K)

    def _fits(dd):
        zrq = ((n // _NS) + _K - 1) // _K * _K
        return (_NS * zrq + n) * dd + _NS * (2 * _K * dd + 16 * _K) <= 2097151

    h = node_feat
    layers = params['layers']
    for i, lp in enumerate(layers):
        d = h.shape[1]
        halves = (not _fits(d)) and d % 2 == 0 and _fits(d // 2)
        if halves:
            h2 = jnp.transpose(h.reshape(n, 2, d // 2), (1, 0, 2))
            parts = _make_agg(n, d // 2, _NC * c, True)(h2, src16, dst16)
        else:
            parts = _make_agg(n, d, c)(h, src32, dst32)
        hd = lp['W1'].shape[1]
        args = (lp['eps'].reshape(1), h, parts, lp['W1'],
                lp['b1'].reshape(1, -1), lp['gamma'].reshape(1, -1),
                lp['beta'].reshape(1, -1), lp['W2'], lp['b2'].reshape(1, -1))
        if i + 1 < len(layers):
            h = pl.pallas_call(
                functools.partial(_mlp_body, halves),
                out_shape=jax.ShapeDtypeStruct((n, hd), jnp.float32),
                in_specs=_specs(8),
            )(*args)
        else:
            out_d = params['Wo'].shape[1]
            h = pl.pallas_call(
                functools.partial(_mlp_out_body, halves),
                out_shape=jax.ShapeDtypeStruct((n, out_d), jnp.float32),
                in_specs=_specs(10),
            )(*args, params['Wo'], params['bo'].reshape(1, -1))
    return h
